# Initial kernel scaffold; baseline (speedup 1.0000x reference)
import jax
import jax.numpy as jnp
from jax.experimental import pallas as pl


def kernel(features, idx):
    raise NotImplementedError("write your pallas kernel here")


# speedup vs baseline: 11.2956x; 11.2956x over previous
import functools

import jax
import jax.numpy as jnp
from jax import lax
from jax.experimental import pallas as pl
from jax.experimental.pallas import tpu as pltpu
from jax.experimental.pallas import tpu_sc as plsc

B, C, N = 8, 64, 16384
P, S = 2048, 32
J = P * S
NW = 32
WPB = NW // B
CPW = C // WPB
CBLK = 4
JC = 4096

_mesh = plsc.VectorSubcoreMesh(core_axis_name="c", subcore_axis_name="s")


@functools.partial(
    pl.kernel,
    mesh=_mesh,
    out_type=jax.ShapeDtypeStruct((B, C, J), jnp.float32),
    scratch_types=[
        pltpu.VMEM((CBLK * N,), jnp.float32),
        pltpu.VMEM((JC,), jnp.int32),
        pltpu.VMEM((CBLK, JC), jnp.float32),
    ],
    compiler_params=pltpu.CompilerParams(needs_layout_passes=False),
)
def _group_sc(feat_hbm, idx_hbm, out_hbm, feat_v, idx_v, out_v):
    cid = lax.axis_index("c")
    sid = lax.axis_index("s")
    w = sid * 2 + cid
    b = w // WPB
    c0 = (w % WPB) * CPW

    def sweep(cg, _):
        cbase = c0 + cg * CBLK
        for cc in range(CBLK):
            pltpu.sync_copy(feat_hbm.at[b, cbase + cc, :],
                            feat_v.at[pl.ds(cc * N, N)])

        def chunk(jc, _):
            j0 = jc * JC
            pltpu.sync_copy(idx_hbm.at[b, pl.ds(j0, JC)], idx_v)

            def gather(g, _):
                iv = idx_v[pl.ds(g * 16, 16)]
                for cc in range(CBLK):
                    out_v[cc, pl.ds(g * 16, 16)] = plsc.load_gather(
                        feat_v, [iv + cc * N])
                return 0

            lax.fori_loop(0, JC // 16, gather, 0, unroll=4)
            for cc in range(CBLK):
                pltpu.sync_copy(out_v.at[cc],
                                out_hbm.at[b, cbase + cc, pl.ds(j0, JC)])
            return 0

        lax.fori_loop(0, J // JC, chunk, 0)
        return 0

    lax.fori_loop(0, CPW // CBLK, sweep, 0)


def kernel(features, idx):
    idx32 = idx.reshape(B, J).astype(jnp.int32)
    out = _group_sc(features, idx32)
    return out.reshape(B, C, P, S)


# speedup vs baseline: 22.3296x; 1.9768x over previous
import functools

import jax
import jax.numpy as jnp
from jax import lax
from jax.experimental import pallas as pl
from jax.experimental.pallas import tpu as pltpu
from jax.experimental.pallas import tpu_sc as plsc

B, C, N = 8, 64, 16384
P, S = 2048, 32
J = P * S
NW = 32
WPB = NW // B
CPW = C // WPB
CBLK = 4
NSWEEP = CPW // CBLK
JC = 4096
NCH = J // JC
T = NSWEEP * NCH

_mesh = plsc.VectorSubcoreMesh(core_axis_name="c", subcore_axis_name="s")


@functools.partial(
    pl.kernel,
    mesh=_mesh,
    out_type=jax.ShapeDtypeStruct((B, C, J), jnp.float32),
    scratch_types=[
        pltpu.VMEM((CBLK, N), jnp.float32),
        pltpu.VMEM((2, JC), jnp.int32),
        pltpu.VMEM((2, CBLK, JC), jnp.float32),
        pltpu.SemaphoreType.DMA((2,)),
        pltpu.SemaphoreType.DMA((2,)),
        pltpu.SemaphoreType.DMA,
    ],
    compiler_params=pltpu.CompilerParams(needs_layout_passes=False),
)
def _group_sc(feat_hbm, idx_hbm, out_hbm, feat_v, idx_v, out_v,
              isem, osem, fsem):
    cid = lax.axis_index("c")
    sid = lax.axis_index("s")
    w = sid * 2 + cid
    b = w // WPB
    c0 = (w % WPB) * CPW

    def idx_copy(t, buf):
        jc = lax.rem(t, NCH)
        return pltpu.make_async_copy(
            idx_hbm.at[b, pl.ds(jc * JC, JC)], idx_v.at[buf], isem.at[buf])

    def out_copy(t, buf):
        cbase = c0 + (t // NCH) * CBLK
        jc = lax.rem(t, NCH)
        return pltpu.make_async_copy(
            out_v.at[buf],
            out_hbm.at[b, pl.ds(cbase, CBLK), pl.ds(jc * JC, JC)],
            osem.at[buf])

    def do_chunk(tp, t, buf):
        idx_copy(t, buf).wait()
        @pl.when(t + 1 < T)
        def _():
            idx_copy(t + 1, 1 - buf).start()
        @pl.when(tp > 0)
        def _():
            out_copy(t - 2, buf).wait()

        @plsc.parallel_loop(0, JC // 16, unroll=4)
        def _gather(g):
            iv = idx_v[buf, pl.ds(g * 16, 16)]
            for cc in range(CBLK):
                out_v[buf, cc, pl.ds(g * 16, 16)] = plsc.load_gather(
                    feat_v, [jnp.full((16,), cc, jnp.int32), iv])

        out_copy(t, buf).start()

    def feat_copy(sweep):
        cbase = c0 + sweep * CBLK
        return pltpu.make_async_copy(
            feat_hbm.at[b, pl.ds(cbase, CBLK), :], feat_v, fsem)

    idx_copy(0, 0).start()

    def pair(tp, _):
        @pl.when(lax.rem(tp, T // (2 * NSWEEP)) == 0)
        def _():
            fc = feat_copy(tp // (T // (2 * NSWEEP)))
            fc.start()
            fc.wait()

        do_chunk(tp, 2 * tp, 0)
        do_chunk(tp, 2 * tp + 1, 1)
        return 0

    lax.fori_loop(0, T // 2, pair, 0)

    out_copy(T - 2, 0).wait()
    out_copy(T - 1, 1).wait()


def kernel(features, idx):
    idx32 = idx.reshape(B, J).astype(jnp.int32)
    out = _group_sc(features, idx32)
    return out.reshape(B, C, P, S)
